# baseline (speedup 1.0000x reference)
import dataclasses
import functools

import jax
import jax.numpy as jnp
from jax import lax
from jax.experimental import pallas as pl
from jax.experimental.pallas import tpu as pltpu
from jax.experimental.pallas import tpu_sc as plsc

NC = 2
NS = 16
NW = NC * NS
L = 16
D = 64
GAMMA = 12.0

CHUNK = 128


def _sc_score(src, rel, dst, ent_embed, rel_embed):
    batch = src.shape[0]
    per_w = batch // NW
    nchunk = per_w // CHUNK
    mesh = plsc.VectorSubcoreMesh(core_axis_name="c", subcore_axis_name="s")
    cp = pltpu.CompilerParams()
    if "needs_layout_passes" in pltpu.CompilerParams.__dataclass_fields__:
        cp = dataclasses.replace(cp, needs_layout_passes=False)
    cp = dataclasses.replace(cp, use_tc_tiling_on_sc=False)

    @functools.partial(
        pl.kernel,
        out_type=jax.ShapeDtypeStruct((batch,), jnp.float32),
        mesh=mesh,
        compiler_params=cp,
        scratch_types=[
            pltpu.VMEM((CHUNK,), jnp.int32),
            pltpu.VMEM((CHUNK,), jnp.int32),
            pltpu.VMEM((CHUNK,), jnp.int32),
            pltpu.VMEM((CHUNK, D), jnp.float32),
            pltpu.VMEM((CHUNK, D), jnp.float32),
            pltpu.VMEM((CHUNK, D), jnp.float32),
            pltpu.VMEM((CHUNK,), jnp.float32),
            pltpu.SemaphoreType.DMA,
            pltpu.SemaphoreType.DMA,
            pltpu.SemaphoreType.DMA,
        ],
    )
    def sc_kernel(src_hbm, rel_hbm, dst_hbm, ent_hbm, relt_hbm, out_hbm,
                  si_v, di_v, ri_v, h_v, t_v, r_v, s_v, sem_h, sem_t, sem_r):
        wid = lax.axis_index("s") * NC + lax.axis_index("c")
        base = wid * per_w

        @pl.loop(0, nchunk)
        def _chunk(k):
            off = base + k * CHUNK
            pltpu.sync_copy(src_hbm.at[pl.ds(off, CHUNK)], si_v)
            pltpu.sync_copy(dst_hbm.at[pl.ds(off, CHUNK)], di_v)
            pltpu.sync_copy(rel_hbm.at[pl.ds(off, CHUNK)], ri_v)
            cp_h = pltpu.async_copy(ent_hbm.at[si_v], h_v, sem_h)
            cp_t = pltpu.async_copy(ent_hbm.at[di_v], t_v, sem_t)
            cp_r = pltpu.async_copy(relt_hbm.at[ri_v], r_v, sem_r)
            cp_h.wait()
            cp_t.wait()
            cp_r.wait()

            lane = lax.iota(jnp.int32, L)

            @pl.loop(0, CHUNK // L)
            def _group(g):
                vec = jnp.zeros((L,), jnp.float32)
                for j in range(L):
                    row = g * L + j
                    acc = jnp.zeros((L,), jnp.float32)
                    for c in range(D // L):
                        hv = h_v[row, pl.ds(c * L, L)]
                        tv = t_v[row, pl.ds(c * L, L)]
                        rv = r_v[row, pl.ds(c * L, L)]
                        acc = acc + jnp.abs(hv + rv - tv)
                    vec = jnp.where(lane == j, GAMMA - jnp.sum(acc), vec)
                s_v[pl.ds(g * L, L)] = vec

            pltpu.sync_copy(s_v, out_hbm.at[pl.ds(off, CHUNK)])

    return sc_kernel(src, rel, dst, ent_embed, rel_embed)


def kernel(src, rel, dst, mode, ent_embed, rel_embed):
    del mode
    return _sc_score(src, rel, dst, ent_embed, rel_embed)
